# baseline (speedup 1.0000x reference)
import jax
import jax.numpy as jnp
from jax import lax
from jax.experimental import pallas as pl
from jax.experimental.pallas import tpu as pltpu
from jax.experimental.pallas import tpu_sc as plsc

_RELS = [("point", "point"), ("point", "line"), ("point", "polygon"),
         ("line", "line"), ("line", "point"), ("line", "polygon"),
         ("polygon", "polygon"), ("polygon", "point"), ("polygon", "line")]
_TYPES = ["point", "line", "polygon"]
_N = 10000
_NP = 10240
_E = 160000
_EP = 163840
_D = 512
_DO = 128
_DH = 64
_NC = 2
_NS = 16
_EPT = _EP // _NS
_CH = 128
_NCHUNK = _EPT // _CH
_HCHUNK = _NCHUNK // 2
_NB = 3
_RPT = _NP // _NS
_ROWB = 400


def _mm_h_body(x_ref, w_ref, h_ref):
    h_ref[0, 0] = jnp.dot(x_ref[0], w_ref[0, 0],
                          preferred_element_type=jnp.float32)


def _stage1_h(xs, wf):
    return pl.pallas_call(
        _mm_h_body,
        grid=(3, _N // _ROWB, 3, _NC),
        in_specs=[
            pl.BlockSpec((1, _ROWB, _D), lambda t, i, j, c: (t, i, 0)),
            pl.BlockSpec((1, 1, _D, _DH),
                         lambda t, i, j, c: (t, 2 * j + c, 0, 0)),
        ],
        out_specs=pl.BlockSpec((1, 1, _ROWB, _DH),
                               lambda t, i, j, c: (c, 3 * t + j, i, 0)),
        out_shape=jax.ShapeDtypeStruct((_NC, 9, _NP, _DH), jnp.float32),
    )(xs, wf)


def _mm_a_body(x_ref, u_ref, a_ref):
    a_ref[0] = jnp.dot(x_ref[0], u_ref[0], preferred_element_type=jnp.float32)


def _stage1_a(xs, us):
    return pl.pallas_call(
        _mm_a_body,
        grid=(3, _N // _ROWB),
        in_specs=[
            pl.BlockSpec((1, _ROWB, _D), lambda t, i: (t, i, 0)),
            pl.BlockSpec((1, _D, _DO), lambda t, i: (t, 0, 0)),
        ],
        out_specs=pl.BlockSpec((1, _ROWB, _DO), lambda t, i: (t, i, 0)),
        out_shape=jax.ShapeDtypeStruct((3, _N, _DO), jnp.float32),
    )(xs, us)


def _sc_body(src_h, dst_h, asrc_h, adst_h, h_h, acc_out, den_out,
             asrc_v, adst_v, srcb, dstb, rowb, zbuf, zden, denloc, iot,
             mbuf, acc_s, den_s, zsem, gs0, gs1, gs2, ss0, ss1, ss2):
    c = lax.axis_index("c")
    s = lax.axis_index("s")
    gsem = [gs0, gs1, gs2]
    ssem = [ss0, ss1, ss2]
    zvec = jnp.zeros((16,), jnp.float32)
    lane = lax.iota(jnp.int32, 16)

    def _zb(i, _):
        for q in range(4):
            zbuf[i, pl.ds(q * 16, 16)] = zvec
        return 0
    lax.fori_loop(0, 128, _zb, 0)

    def _zd(i, _):
        zden[i, :] = zvec
        return 0
    lax.fori_loop(0, 40, _zd, 0)

    def _io(i, _):
        k = i // 8
        j = i % 8
        iot[k, pl.ds(j * 16, 16)] = lane + i * 16
        return 0
    lax.fori_loop(0, 40, _io, 0)

    def _rel(r, _carry):
        pltpu.sync_copy(asrc_h.at[pl.ds(r * _NP, _NP)], asrc_v)
        pltpu.sync_copy(adst_h.at[pl.ds(r * _NP, _NP)], adst_v)

        zcps = [pltpu.async_copy(
            zbuf, acc_s.at[pl.ds(s * _RPT + k * 128, 128)], zsem)
            for k in range(_RPT // 128)]
        for cp in zcps:
            cp.wait()
        pltpu.sync_copy(zden, den_s.at[pl.ds(s * 40, 40)])

        def _zl(i, _):
            denloc[i, :] = zvec
            return 0
        lax.fori_loop(0, _NP // 16, _zl, 0)

        def _mxs(i, m):
            return jnp.maximum(m, asrc_v[pl.ds(i * 16, 16)])
        def _mxd(i, m):
            return jnp.maximum(m, adst_v[pl.ds(i * 16, 16)])
        neg = jnp.full((16,), -1e30, jnp.float32)

        def _lanemax(m):
            for sh in (1, 2, 4, 8):
                mbuf[...] = m
                m = jnp.maximum(m, plsc.load_gather(
                    mbuf, [jnp.bitwise_xor(lane, sh)]))
            return m
        mtot = _lanemax(lax.fori_loop(0, _NP // 16, _mxs, neg)) + \
               _lanemax(lax.fori_loop(0, _NP // 16, _mxd, neg))
        m_sh = jnp.where(mtot >= 0, mtot, 0.2 * mtot)

        plsc.subcore_barrier()

        def _compute(k, b):
            for i in range(8):
                sv = srcb[k, pl.ds(i * 16, 16)]
                dv = dstb[k, pl.ds(i * 16, 16)]
                a = plsc.load_gather(asrc_v, [sv]) + \
                    plsc.load_gather(adst_v, [dv])
                a = jnp.where(a >= 0, a, 0.2 * a)
                ev = jnp.exp(a - m_sh)
                plsc.addupdate_scatter(
                    denloc,
                    [jnp.right_shift(dv, 4), jnp.bitwise_and(dv, 15)], ev)
                for j2 in range(16):
                    e = ev[j2]
                    ro = i * 16 + j2
                    for q in range(4):
                        sl = pl.ds(q * 16, 16)
                        rowb[b, ro, sl] = rowb[b, ro, sl] * e

        def _half(h, _):
            pltpu.sync_copy(src_h.at[r, s, pl.ds(h * _HCHUNK, _HCHUNK)], srcb)
            pltpu.sync_copy(dst_h.at[r, s, pl.ds(h * _HCHUNK, _HCHUNK)], dstb)

            pltpu.async_copy(h_h.at[c, r].at[srcb.at[0]], rowb.at[0], gsem[0])

            def _chunk(k, _k):
                bk = lax.rem(k, _NB)
                bk1 = lax.rem(k + 1, _NB)

                @pl.when(k >= 2)
                def _():
                    for b in range(_NB):
                        @pl.when(bk1 == b)
                        def _():
                            pltpu.make_async_copy(
                                rowb.at[b], acc_s.at[dstb.at[k]],
                                ssem[b]).wait()

                @pl.when(k < _HCHUNK - 1)
                def _():
                    for b in range(_NB):
                        @pl.when(bk1 == b)
                        def _():
                            pltpu.async_copy(
                                h_h.at[c, r].at[srcb.at[k + 1]], rowb.at[b],
                                gsem[b])
                for b in range(_NB):
                    @pl.when(bk == b)
                    def _():
                        pltpu.make_async_copy(
                            h_h.at[c, r].at[srcb.at[k]], rowb.at[b],
                            gsem[b]).wait()
                _compute(k, bk)
                for b in range(_NB):
                    @pl.when(bk == b)
                    def _():
                        pltpu.async_copy(
                            rowb.at[b], acc_s.at[dstb.at[k]], ssem[b],
                            add=True)
                return 0
            lax.fori_loop(0, _HCHUNK, _chunk, 0)
            for kk in (_HCHUNK - 2, _HCHUNK - 1):
                b = kk % _NB
                pltpu.make_async_copy(
                    rowb.at[b], acc_s.at[dstb.at[kk]], ssem[b]).wait()
            return 0
        lax.fori_loop(0, 2, _half, 0)

        @pl.when(c == 0)
        def _():
            for k in range(5):
                pltpu.sync_copy(denloc.at[pl.ds(k * 128, 128)],
                                den_s.at[iot.at[k]], add=True)

        plsc.subcore_barrier()

        nout = _RPT // 128
        pltpu.async_copy(acc_s.at[pl.ds(s * _RPT, 128)], rowb.at[0], gsem[0])
        for k in range(nout):
            b = k % _NB
            b1 = (k + 1) % _NB
            pltpu.make_async_copy(
                acc_s.at[pl.ds(s * _RPT + k * 128, 128)], rowb.at[b],
                gsem[b]).wait()
            if k >= 2:
                pltpu.make_async_copy(
                    rowb.at[b1],
                    acc_out.at[c, r, pl.ds(s * _RPT + (k - 2) * 128, 128), :],
                    ssem[b1]).wait()
            if k < nout - 1:
                pltpu.async_copy(
                    acc_s.at[pl.ds(s * _RPT + (k + 1) * 128, 128)],
                    rowb.at[b1], gsem[b1])
            pltpu.async_copy(
                rowb.at[b],
                acc_out.at[c, r, pl.ds(s * _RPT + k * 128, 128), :], ssem[b])
        for kk in (nout - 2, nout - 1):
            b = kk % _NB
            pltpu.make_async_copy(
                rowb.at[b],
                acc_out.at[c, r, pl.ds(s * _RPT + kk * 128, 128), :],
                ssem[b]).wait()

        @pl.when(jnp.logical_and(c == 0, s == 0))
        def _():
            pltpu.sync_copy(den_s, denloc)
            pltpu.sync_copy(denloc, den_out.at[r])

        plsc.subcore_barrier()
        return 0

    lax.fori_loop(0, 9, _rel, 0)


def _stage2(src_p, dst_p, asrc_p, adst_p, h_p):
    mesh = plsc.VectorSubcoreMesh(core_axis_name="c", subcore_axis_name="s",
                                  num_cores=_NC, num_subcores=_NS)
    return pl.kernel(
        _sc_body,
        out_type=[
            jax.ShapeDtypeStruct((_NC, 9, _NP, _DH), jnp.float32),
            jax.ShapeDtypeStruct((9, _NP // 16, 16), jnp.float32),
        ],
        mesh=mesh,
        compiler_params=pltpu.CompilerParams(needs_layout_passes=False,
                                             use_tc_tiling_on_sc=False),
        scratch_types=[
            pltpu.VMEM((_NP,), jnp.float32),
            pltpu.VMEM((_NP,), jnp.float32),
            pltpu.VMEM((_HCHUNK, _CH), jnp.int32),
            pltpu.VMEM((_HCHUNK, _CH), jnp.int32),
            pltpu.VMEM((_NB, _CH, _DH), jnp.float32),
            pltpu.VMEM((128, _DH), jnp.float32),
            pltpu.VMEM((40, 16), jnp.float32),
            pltpu.VMEM((_NP // 16, 16), jnp.float32),
            pltpu.VMEM((5, 128), jnp.int32),
            pltpu.VMEM((16,), jnp.float32),
            pltpu.VMEM_SHARED((_NP, _DH), jnp.float32),
            pltpu.VMEM_SHARED((_NP // 16, 16), jnp.float32),
            pltpu.SemaphoreType.DMA,
            pltpu.SemaphoreType.DMA,
            pltpu.SemaphoreType.DMA,
            pltpu.SemaphoreType.DMA,
            pltpu.SemaphoreType.DMA,
            pltpu.SemaphoreType.DMA,
            pltpu.SemaphoreType.DMA,
        ],
    )(src_p, dst_p, asrc_p, adst_p, h_p)


def _comb_body(acc_ref, den_ref, bias_ref, op_ref, ol_ref, og_ref):
    acc = acc_ref[...]
    dn = den_ref[...]
    bias = bias_ref[...]

    def _mix(rels):
        out = 0.0
        for r in rels:
            ar = jnp.concatenate([acc[0, r], acc[1, r]], axis=-1)
            out = out + ar / (dn[:, r][:, None] + 1e-16) + bias[r][None, :]
        return jnp.maximum(out * (1.0 / 3.0), 0.0)

    op_ref[...] = _mix([0, 4, 7])
    ol_ref[...] = _mix([1, 3, 8])
    og_ref[...] = _mix([2, 5, 6])


def _stage3(acc, den, bias):
    return pl.pallas_call(
        _comb_body,
        grid=(_N // _ROWB,),
        in_specs=[
            pl.BlockSpec((_NC, 9, _ROWB, _DH), lambda i: (0, 0, i, 0)),
            pl.BlockSpec((_ROWB, 9), lambda i: (i, 0)),
            pl.BlockSpec((9, _DO), lambda i: (0, 0)),
        ],
        out_specs=[
            pl.BlockSpec((_ROWB, _DO), lambda i: (i, 0)),
            pl.BlockSpec((_ROWB, _DO), lambda i: (i, 0)),
            pl.BlockSpec((_ROWB, _DO), lambda i: (i, 0)),
        ],
        out_shape=[jax.ShapeDtypeStruct((_N, _DO), jnp.float32)] * 3,
    )(acc, den, bias)


def kernel(x_point, x_line, x_polygon, params,
           ei_point_point, ei_point_line, ei_point_polygon,
           ei_line_line, ei_line_point, ei_line_polygon,
           ei_polygon_polygon, ei_polygon_point, ei_polygon_line):
    eis = {("point", "point"): ei_point_point,
           ("point", "line"): ei_point_line,
           ("point", "polygon"): ei_point_polygon,
           ("line", "line"): ei_line_line,
           ("line", "point"): ei_line_point,
           ("line", "polygon"): ei_line_polygon,
           ("polygon", "polygon"): ei_polygon_polygon,
           ("polygon", "point"): ei_polygon_point,
           ("polygon", "line"): ei_polygon_line}
    xd = {"point": x_point, "line": x_line, "polygon": x_polygon}

    src_rels = {t: [i for i, (sr, _) in enumerate(_RELS) if sr == t] for t in _TYPES}
    dst_rels = {t: [i for i, (_, dr) in enumerate(_RELS) if dr == t] for t in _TYPES}

    wfs, uss = [], []
    for t in _TYPES:
        cols = [params["%s__%s" % _RELS[r]]["W"] for r in src_rels[t]]
        wcat = jnp.concatenate(cols, axis=1)
        wfs.append(jnp.transpose(wcat.reshape(_D, 6, _DH), (1, 0, 2)))
        ucols = [(params["%s__%s" % _RELS[r]]["W"]
                  @ params["%s__%s" % _RELS[r]]["att_src"])[:, None]
                 for r in src_rels[t]]
        ucols += [(params["%s__%s" % _RELS[r]]["W"]
                   @ params["%s__%s" % _RELS[r]]["att_dst"])[:, None]
                  for r in dst_rels[t]]
        u = jnp.concatenate(ucols, axis=1)
        uss.append(jnp.pad(u, ((0, 0), (0, _DO - u.shape[1]))))
    xs = jnp.stack([xd[t] for t in _TYPES])

    h_p = _stage1_h(xs, jnp.stack(wfs))
    a_all = _stage1_a(xs, jnp.stack(uss))

    ti = {t: i for i, t in enumerate(_TYPES)}
    asrc_list, adst_list = [None] * 9, [None] * 9
    for t in _TYPES:
        for j, r in enumerate(src_rels[t]):
            asrc_list[r] = a_all[ti[t], :, j]
        for j, r in enumerate(dst_rels[t]):
            adst_list[r] = a_all[ti[t], :, 3 + j]

    pad_n = _NP - _N
    asrc_p = jnp.stack([jnp.pad(a, (0, pad_n), constant_values=-1e30)
                        for a in asrc_list])
    adst_p = jnp.stack([jnp.pad(a, (0, pad_n)) for a in adst_list])
    src_p = jnp.stack([jnp.pad(eis[rel][0], (0, _EP - _E), constant_values=_N)
                       for rel in _RELS]).reshape(9, _NS, _NCHUNK, _CH)
    dst_p = jnp.stack([jnp.pad(eis[rel][1], (0, _EP - _E), constant_values=_N)
                       for rel in _RELS]).reshape(9, _NS, _NCHUNK, _CH)

    acc, den = _stage2(src_p, dst_p, asrc_p.reshape(-1), adst_p.reshape(-1),
                       h_p)

    bias = jnp.stack([params["%s__%s" % rel]["bias"] for rel in _RELS])
    return _stage3(acc, den.reshape(9, _NP).T, bias)


# speedup vs baseline: 1.0019x; 1.0019x over previous
import jax
import jax.numpy as jnp
from jax import lax
from jax.experimental import pallas as pl
from jax.experimental.pallas import tpu as pltpu
from jax.experimental.pallas import tpu_sc as plsc

_RELS = [("point", "point"), ("point", "line"), ("point", "polygon"),
         ("line", "line"), ("line", "point"), ("line", "polygon"),
         ("polygon", "polygon"), ("polygon", "point"), ("polygon", "line")]
_TYPES = ["point", "line", "polygon"]
_N = 10000
_NP = 10240
_E = 160000
_EP = 163840
_D = 512
_DO = 128
_DH = 64
_NC = 2
_NS = 16
_EPT = _EP // _NS
_CH = 128
_NCHUNK = _EPT // _CH
_HCHUNK = _NCHUNK // 2
_NB = 4
_RPT = _NP // _NS
_ROWB = 400


def _mm_h_body(x_ref, w_ref, h_ref):
    h_ref[0, 0] = jnp.dot(x_ref[0], w_ref[0, 0],
                          preferred_element_type=jnp.float32)


def _stage1_h(xs, wf):
    return pl.pallas_call(
        _mm_h_body,
        grid=(3, _N // _ROWB, 3, _NC),
        in_specs=[
            pl.BlockSpec((1, _ROWB, _D), lambda t, i, j, c: (t, i, 0)),
            pl.BlockSpec((1, 1, _D, _DH),
                         lambda t, i, j, c: (t, 2 * j + c, 0, 0)),
        ],
        out_specs=pl.BlockSpec((1, 1, _ROWB, _DH),
                               lambda t, i, j, c: (c, 3 * t + j, i, 0)),
        out_shape=jax.ShapeDtypeStruct((_NC, 9, _NP, _DH), jnp.float32),
    )(xs, wf)


def _mm_a_body(x_ref, u_ref, a_ref):
    a_ref[0] = jnp.dot(x_ref[0], u_ref[0], preferred_element_type=jnp.float32)


def _stage1_a(xs, us):
    return pl.pallas_call(
        _mm_a_body,
        grid=(3, _N // _ROWB),
        in_specs=[
            pl.BlockSpec((1, _ROWB, _D), lambda t, i: (t, i, 0)),
            pl.BlockSpec((1, _D, _DO), lambda t, i: (t, 0, 0)),
        ],
        out_specs=pl.BlockSpec((1, _ROWB, _DO), lambda t, i: (t, i, 0)),
        out_shape=jax.ShapeDtypeStruct((3, _N, _DO), jnp.float32),
    )(xs, us)


def _sc_body(src_h, dst_h, asrc_h, adst_h, h_h, acc_out, den_out,
             asrc_v, adst_v, srcb, dstb, rowb, zbuf, zden, denloc, iot,
             mbuf, acc_s, den_s, zsem, gs0, gs1, gs2, gs3,
             ss0, ss1, ss2, ss3):
    c = lax.axis_index("c")
    s = lax.axis_index("s")
    gsem = [gs0, gs1, gs2, gs3]
    ssem = [ss0, ss1, ss2, ss3]
    zvec = jnp.zeros((16,), jnp.float32)
    lane = lax.iota(jnp.int32, 16)

    def _zb(i, _):
        for q in range(4):
            zbuf[i, pl.ds(q * 16, 16)] = zvec
        return 0
    lax.fori_loop(0, 128, _zb, 0)

    def _zd(i, _):
        zden[i, :] = zvec
        return 0
    lax.fori_loop(0, 40, _zd, 0)

    def _io(i, _):
        k = i // 8
        j = i % 8
        iot[k, pl.ds(j * 16, 16)] = lane + i * 16
        return 0
    lax.fori_loop(0, 40, _io, 0)

    def _rel(r, _carry):
        pltpu.sync_copy(asrc_h.at[pl.ds(r * _NP, _NP)], asrc_v)
        pltpu.sync_copy(adst_h.at[pl.ds(r * _NP, _NP)], adst_v)

        zcps = [pltpu.async_copy(
            zbuf, acc_s.at[pl.ds(s * _RPT + k * 128, 128)], zsem)
            for k in range(_RPT // 128)]
        for cp in zcps:
            cp.wait()
        pltpu.sync_copy(zden, den_s.at[pl.ds(s * 40, 40)])

        def _zl(i, _):
            denloc[i, :] = zvec
            return 0
        lax.fori_loop(0, _NP // 16, _zl, 0)

        def _mxs(i, m):
            return jnp.maximum(m, asrc_v[pl.ds(i * 16, 16)])
        def _mxd(i, m):
            return jnp.maximum(m, adst_v[pl.ds(i * 16, 16)])
        neg = jnp.full((16,), -1e30, jnp.float32)

        def _lanemax(m):
            for sh in (1, 2, 4, 8):
                mbuf[...] = m
                m = jnp.maximum(m, plsc.load_gather(
                    mbuf, [jnp.bitwise_xor(lane, sh)]))
            return m
        mtot = _lanemax(lax.fori_loop(0, _NP // 16, _mxs, neg)) + \
               _lanemax(lax.fori_loop(0, _NP // 16, _mxd, neg))
        m_sh = jnp.where(mtot >= 0, mtot, 0.2 * mtot)

        plsc.subcore_barrier()

        def _compute(k, b):
            for i in range(8):
                sv = srcb[k, pl.ds(i * 16, 16)]
                dv = dstb[k, pl.ds(i * 16, 16)]
                a = plsc.load_gather(asrc_v, [sv]) + \
                    plsc.load_gather(adst_v, [dv])
                a = jnp.where(a >= 0, a, 0.2 * a)
                ev = jnp.exp(a - m_sh)
                plsc.addupdate_scatter(
                    denloc,
                    [jnp.right_shift(dv, 4), jnp.bitwise_and(dv, 15)], ev)
                for j2 in range(16):
                    e = ev[j2]
                    ro = i * 16 + j2
                    for q in range(4):
                        sl = pl.ds(q * 16, 16)
                        rowb[b, ro, sl] = rowb[b, ro, sl] * e

        def _half(h, _):
            pltpu.sync_copy(src_h.at[r, s, pl.ds(h * _HCHUNK, _HCHUNK)], srcb)
            pltpu.sync_copy(dst_h.at[r, s, pl.ds(h * _HCHUNK, _HCHUNK)], dstb)

            pltpu.async_copy(h_h.at[c, r].at[srcb.at[0]], rowb.at[0], gsem[0])

            def _chunk(k, _k):
                bk = lax.rem(k, _NB)
                bk1 = lax.rem(k + 1, _NB)

                @pl.when(k >= _NB - 1)
                def _():
                    for b in range(_NB):
                        @pl.when(bk1 == b)
                        def _():
                            pltpu.make_async_copy(
                                rowb.at[b], acc_s.at[dstb.at[k]],
                                ssem[b]).wait()

                @pl.when(k < _HCHUNK - 1)
                def _():
                    for b in range(_NB):
                        @pl.when(bk1 == b)
                        def _():
                            pltpu.async_copy(
                                h_h.at[c, r].at[srcb.at[k + 1]], rowb.at[b],
                                gsem[b])
                for b in range(_NB):
                    @pl.when(bk == b)
                    def _():
                        pltpu.make_async_copy(
                            h_h.at[c, r].at[srcb.at[k]], rowb.at[b],
                            gsem[b]).wait()
                _compute(k, bk)
                for b in range(_NB):
                    @pl.when(bk == b)
                    def _():
                        pltpu.async_copy(
                            rowb.at[b], acc_s.at[dstb.at[k]], ssem[b],
                            add=True)
                return 0
            lax.fori_loop(0, _HCHUNK, _chunk, 0)
            for kk in range(_HCHUNK - _NB + 1, _HCHUNK):
                b = kk % _NB
                pltpu.make_async_copy(
                    rowb.at[b], acc_s.at[dstb.at[kk]], ssem[b]).wait()
            return 0
        lax.fori_loop(0, 2, _half, 0)

        @pl.when(c == 0)
        def _():
            for k in range(5):
                pltpu.sync_copy(denloc.at[pl.ds(k * 128, 128)],
                                den_s.at[iot.at[k]], add=True)

        plsc.subcore_barrier()

        nout = _RPT // 128
        pltpu.async_copy(acc_s.at[pl.ds(s * _RPT, 128)], rowb.at[0], gsem[0])
        for k in range(nout):
            b = k % _NB
            b1 = (k + 1) % _NB
            pltpu.make_async_copy(
                acc_s.at[pl.ds(s * _RPT + k * 128, 128)], rowb.at[b],
                gsem[b]).wait()
            if k >= 2:
                b2 = (k - 2) % _NB
                pltpu.make_async_copy(
                    rowb.at[b2],
                    acc_out.at[c, r, pl.ds(s * _RPT + (k - 2) * 128, 128), :],
                    ssem[b2]).wait()
            if k < nout - 1:
                pltpu.async_copy(
                    acc_s.at[pl.ds(s * _RPT + (k + 1) * 128, 128)],
                    rowb.at[b1], gsem[b1])
            pltpu.async_copy(
                rowb.at[b],
                acc_out.at[c, r, pl.ds(s * _RPT + k * 128, 128), :], ssem[b])
        for kk in (nout - 2, nout - 1):
            b = kk % _NB
            pltpu.make_async_copy(
                rowb.at[b],
                acc_out.at[c, r, pl.ds(s * _RPT + kk * 128, 128), :],
                ssem[b]).wait()

        @pl.when(jnp.logical_and(c == 0, s == 0))
        def _():
            pltpu.sync_copy(den_s, denloc)
            pltpu.sync_copy(denloc, den_out.at[r])

        plsc.subcore_barrier()
        return 0

    lax.fori_loop(0, 9, _rel, 0)


def _stage2(src_p, dst_p, asrc_p, adst_p, h_p):
    mesh = plsc.VectorSubcoreMesh(core_axis_name="c", subcore_axis_name="s",
                                  num_cores=_NC, num_subcores=_NS)
    return pl.kernel(
        _sc_body,
        out_type=[
            jax.ShapeDtypeStruct((_NC, 9, _NP, _DH), jnp.float32),
            jax.ShapeDtypeStruct((9, _NP // 16, 16), jnp.float32),
        ],
        mesh=mesh,
        compiler_params=pltpu.CompilerParams(needs_layout_passes=False,
                                             use_tc_tiling_on_sc=False),
        scratch_types=[
            pltpu.VMEM((_NP,), jnp.float32),
            pltpu.VMEM((_NP,), jnp.float32),
            pltpu.VMEM((_HCHUNK, _CH), jnp.int32),
            pltpu.VMEM((_HCHUNK, _CH), jnp.int32),
            pltpu.VMEM((_NB, _CH, _DH), jnp.float32),
            pltpu.VMEM((128, _DH), jnp.float32),
            pltpu.VMEM((40, 16), jnp.float32),
            pltpu.VMEM((_NP // 16, 16), jnp.float32),
            pltpu.VMEM((5, 128), jnp.int32),
            pltpu.VMEM((16,), jnp.float32),
            pltpu.VMEM_SHARED((_NP, _DH), jnp.float32),
            pltpu.VMEM_SHARED((_NP // 16, 16), jnp.float32),
            pltpu.SemaphoreType.DMA,
            pltpu.SemaphoreType.DMA,
            pltpu.SemaphoreType.DMA,
            pltpu.SemaphoreType.DMA,
            pltpu.SemaphoreType.DMA,
            pltpu.SemaphoreType.DMA,
            pltpu.SemaphoreType.DMA,
            pltpu.SemaphoreType.DMA,
            pltpu.SemaphoreType.DMA,
        ],
    )(src_p, dst_p, asrc_p, adst_p, h_p)


def _comb_body(acc_ref, den_ref, bias_ref, op_ref, ol_ref, og_ref):
    acc = acc_ref[...]
    dn = den_ref[...]
    bias = bias_ref[...]

    def _mix(rels):
        out = 0.0
        for r in rels:
            ar = jnp.concatenate([acc[0, r], acc[1, r]], axis=-1)
            out = out + ar / (dn[:, r][:, None] + 1e-16) + bias[r][None, :]
        return jnp.maximum(out * (1.0 / 3.0), 0.0)

    op_ref[...] = _mix([0, 4, 7])
    ol_ref[...] = _mix([1, 3, 8])
    og_ref[...] = _mix([2, 5, 6])


def _stage3(acc, den, bias):
    return pl.pallas_call(
        _comb_body,
        grid=(_N // _ROWB,),
        in_specs=[
            pl.BlockSpec((_NC, 9, _ROWB, _DH), lambda i: (0, 0, i, 0)),
            pl.BlockSpec((_ROWB, 9), lambda i: (i, 0)),
            pl.BlockSpec((9, _DO), lambda i: (0, 0)),
        ],
        out_specs=[
            pl.BlockSpec((_ROWB, _DO), lambda i: (i, 0)),
            pl.BlockSpec((_ROWB, _DO), lambda i: (i, 0)),
            pl.BlockSpec((_ROWB, _DO), lambda i: (i, 0)),
        ],
        out_shape=[jax.ShapeDtypeStruct((_N, _DO), jnp.float32)] * 3,
    )(acc, den, bias)


def kernel(x_point, x_line, x_polygon, params,
           ei_point_point, ei_point_line, ei_point_polygon,
           ei_line_line, ei_line_point, ei_line_polygon,
           ei_polygon_polygon, ei_polygon_point, ei_polygon_line):
    eis = {("point", "point"): ei_point_point,
           ("point", "line"): ei_point_line,
           ("point", "polygon"): ei_point_polygon,
           ("line", "line"): ei_line_line,
           ("line", "point"): ei_line_point,
           ("line", "polygon"): ei_line_polygon,
           ("polygon", "polygon"): ei_polygon_polygon,
           ("polygon", "point"): ei_polygon_point,
           ("polygon", "line"): ei_polygon_line}
    xd = {"point": x_point, "line": x_line, "polygon": x_polygon}

    src_rels = {t: [i for i, (sr, _) in enumerate(_RELS) if sr == t] for t in _TYPES}
    dst_rels = {t: [i for i, (_, dr) in enumerate(_RELS) if dr == t] for t in _TYPES}

    wfs, uss = [], []
    for t in _TYPES:
        cols = [params["%s__%s" % _RELS[r]]["W"] for r in src_rels[t]]
        wcat = jnp.concatenate(cols, axis=1)
        wfs.append(jnp.transpose(wcat.reshape(_D, 6, _DH), (1, 0, 2)))
        ucols = [(params["%s__%s" % _RELS[r]]["W"]
                  @ params["%s__%s" % _RELS[r]]["att_src"])[:, None]
                 for r in src_rels[t]]
        ucols += [(params["%s__%s" % _RELS[r]]["W"]
                   @ params["%s__%s" % _RELS[r]]["att_dst"])[:, None]
                  for r in dst_rels[t]]
        u = jnp.concatenate(ucols, axis=1)
        uss.append(jnp.pad(u, ((0, 0), (0, _DO - u.shape[1]))))
    xs = jnp.stack([xd[t] for t in _TYPES])

    h_p = _stage1_h(xs, jnp.stack(wfs))
    a_all = _stage1_a(xs, jnp.stack(uss))

    ti = {t: i for i, t in enumerate(_TYPES)}
    asrc_list, adst_list = [None] * 9, [None] * 9
    for t in _TYPES:
        for j, r in enumerate(src_rels[t]):
            asrc_list[r] = a_all[ti[t], :, j]
        for j, r in enumerate(dst_rels[t]):
            adst_list[r] = a_all[ti[t], :, 3 + j]

    pad_n = _NP - _N
    asrc_p = jnp.stack([jnp.pad(a, (0, pad_n), constant_values=-1e30)
                        for a in asrc_list])
    adst_p = jnp.stack([jnp.pad(a, (0, pad_n)) for a in adst_list])
    src_p = jnp.stack([jnp.pad(eis[rel][0], (0, _EP - _E), constant_values=_N)
                       for rel in _RELS]).reshape(9, _NS, _NCHUNK, _CH)
    dst_p = jnp.stack([jnp.pad(eis[rel][1], (0, _EP - _E), constant_values=_N)
                       for rel in _RELS]).reshape(9, _NS, _NCHUNK, _CH)

    acc, den = _stage2(src_p, dst_p, asrc_p.reshape(-1), adst_p.reshape(-1),
                       h_p)

    bias = jnp.stack([params["%s__%s" % rel]["bias"] for rel in _RELS])
    return _stage3(acc, den.reshape(9, _NP).T, bias)


# speedup vs baseline: 1.1206x; 1.1185x over previous
import jax
import jax.numpy as jnp
from jax import lax
from jax.experimental import pallas as pl
from jax.experimental.pallas import tpu as pltpu
from jax.experimental.pallas import tpu_sc as plsc

_RELS = [("point", "point"), ("point", "line"), ("point", "polygon"),
         ("line", "line"), ("line", "point"), ("line", "polygon"),
         ("polygon", "polygon"), ("polygon", "point"), ("polygon", "line")]
_TYPES = ["point", "line", "polygon"]
_N = 10000
_NP = 10240
_E = 160000
_EP = 163840
_D = 512
_DO = 128
_DH = 64
_NC = 2
_NS = 16
_EPT = _EP // _NS
_CH = 128
_NCHUNK = _EPT // _CH
_HCHUNK = _NCHUNK // 2
_NB = 4
_RPT = _NP // _NS
_ROWB = 2000


def _mm_h_body(x_ref, w_ref, h_ref):
    h_ref[0, 0] = jnp.dot(x_ref[0], w_ref[0, 0],
                          preferred_element_type=jnp.float32)


def _stage1_h(xs, wf):
    return pl.pallas_call(
        _mm_h_body,
        grid=(3, _N // _ROWB, 3, _NC),
        in_specs=[
            pl.BlockSpec((1, _ROWB, _D), lambda t, i, j, c: (t, i, 0)),
            pl.BlockSpec((1, 1, _D, _DH),
                         lambda t, i, j, c: (t, 2 * j + c, 0, 0)),
        ],
        out_specs=pl.BlockSpec((1, 1, _ROWB, _DH),
                               lambda t, i, j, c: (c, 3 * t + j, i, 0)),
        out_shape=jax.ShapeDtypeStruct((_NC, 9, _NP, _DH), jnp.float32),
    )(xs, wf)


def _mm_a_body(x_ref, u_ref, a_ref):
    a_ref[0] = jnp.dot(x_ref[0], u_ref[0], preferred_element_type=jnp.float32)


def _stage1_a(xs, us):
    return pl.pallas_call(
        _mm_a_body,
        grid=(3, _N // _ROWB),
        in_specs=[
            pl.BlockSpec((1, _ROWB, _D), lambda t, i: (t, i, 0)),
            pl.BlockSpec((1, _D, _DO), lambda t, i: (t, 0, 0)),
        ],
        out_specs=pl.BlockSpec((1, _ROWB, _DO), lambda t, i: (t, i, 0)),
        out_shape=jax.ShapeDtypeStruct((3, _N, _DO), jnp.float32),
    )(xs, us)


def _sc_body(src_h, dst_h, asrc_h, adst_h, h_h, acc_out, den_out,
             asrc_v, adst_v, srcb, dstb, rowb, zbuf, zden, denloc, iot,
             mbuf, acc_s, den_s, zsem, gs0, gs1, gs2, gs3,
             ss0, ss1, ss2, ss3):
    c = lax.axis_index("c")
    s = lax.axis_index("s")
    gsem = [gs0, gs1, gs2, gs3]
    ssem = [ss0, ss1, ss2, ss3]
    zvec = jnp.zeros((16,), jnp.float32)
    lane = lax.iota(jnp.int32, 16)

    def _zb(i, _):
        for q in range(4):
            zbuf[i, pl.ds(q * 16, 16)] = zvec
        return 0
    lax.fori_loop(0, 128, _zb, 0)

    def _zd(i, _):
        zden[i, :] = zvec
        return 0
    lax.fori_loop(0, 40, _zd, 0)

    def _io(i, _):
        k = i // 8
        j = i % 8
        iot[k, pl.ds(j * 16, 16)] = lane + i * 16
        return 0
    lax.fori_loop(0, 40, _io, 0)

    def _rel(r, _carry):
        pltpu.sync_copy(asrc_h.at[pl.ds(r * _NP, _NP)], asrc_v)
        pltpu.sync_copy(adst_h.at[pl.ds(r * _NP, _NP)], adst_v)

        zcps = [pltpu.async_copy(
            zbuf, acc_s.at[pl.ds(s * _RPT + k * 128, 128)], zsem)
            for k in range(_RPT // 128)]
        for cp in zcps:
            cp.wait()
        pltpu.sync_copy(zden, den_s.at[pl.ds(s * 40, 40)])

        def _zl(i, _):
            denloc[i, :] = zvec
            return 0
        lax.fori_loop(0, _NP // 16, _zl, 0)

        def _mxs(i, m):
            return jnp.maximum(m, asrc_v[pl.ds(i * 16, 16)])
        def _mxd(i, m):
            return jnp.maximum(m, adst_v[pl.ds(i * 16, 16)])
        neg = jnp.full((16,), -1e30, jnp.float32)

        def _lanemax(m):
            for sh in (1, 2, 4, 8):
                mbuf[...] = m
                m = jnp.maximum(m, plsc.load_gather(
                    mbuf, [jnp.bitwise_xor(lane, sh)]))
            return m
        mtot = _lanemax(lax.fori_loop(0, _NP // 16, _mxs, neg)) + \
               _lanemax(lax.fori_loop(0, _NP // 16, _mxd, neg))
        m_sh = jnp.where(mtot >= 0, mtot, 0.2 * mtot)

        plsc.subcore_barrier()

        def _compute(k, b):
            for i in range(8):
                sv = srcb[k, pl.ds(i * 16, 16)]
                dv = dstb[k, pl.ds(i * 16, 16)]
                a = plsc.load_gather(asrc_v, [sv]) + \
                    plsc.load_gather(adst_v, [dv])
                a = jnp.where(a >= 0, a, 0.2 * a)
                ev = jnp.exp(a - m_sh)
                plsc.addupdate_scatter(
                    denloc,
                    [jnp.right_shift(dv, 4), jnp.bitwise_and(dv, 15)], ev)
                for j2 in range(16):
                    e = ev[j2]
                    ro = i * 16 + j2
                    for q in range(4):
                        sl = pl.ds(q * 16, 16)
                        rowb[b, ro, sl] = rowb[b, ro, sl] * e

        def _half(h, _):
            pltpu.sync_copy(src_h.at[r, s, pl.ds(h * _HCHUNK, _HCHUNK)], srcb)
            pltpu.sync_copy(dst_h.at[r, s, pl.ds(h * _HCHUNK, _HCHUNK)], dstb)

            pltpu.async_copy(h_h.at[c, r].at[srcb.at[0]], rowb.at[0], gsem[0])

            def _chunk(k, _k):
                bk = lax.rem(k, _NB)
                bk1 = lax.rem(k + 1, _NB)

                @pl.when(k >= _NB - 1)
                def _():
                    for b in range(_NB):
                        @pl.when(bk1 == b)
                        def _():
                            pltpu.make_async_copy(
                                rowb.at[b], acc_s.at[dstb.at[k]],
                                ssem[b]).wait()

                @pl.when(k < _HCHUNK - 1)
                def _():
                    for b in range(_NB):
                        @pl.when(bk1 == b)
                        def _():
                            pltpu.async_copy(
                                h_h.at[c, r].at[srcb.at[k + 1]], rowb.at[b],
                                gsem[b])
                for b in range(_NB):
                    @pl.when(bk == b)
                    def _():
                        pltpu.make_async_copy(
                            h_h.at[c, r].at[srcb.at[k]], rowb.at[b],
                            gsem[b]).wait()
                _compute(k, bk)
                for b in range(_NB):
                    @pl.when(bk == b)
                    def _():
                        pltpu.async_copy(
                            rowb.at[b], acc_s.at[dstb.at[k]], ssem[b],
                            add=True)
                return 0
            lax.fori_loop(0, _HCHUNK, _chunk, 0)
            for kk in range(_HCHUNK - _NB + 1, _HCHUNK):
                b = kk % _NB
                pltpu.make_async_copy(
                    rowb.at[b], acc_s.at[dstb.at[kk]], ssem[b]).wait()
            return 0
        lax.fori_loop(0, 2, _half, 0)

        @pl.when(c == 0)
        def _():
            for k in range(5):
                pltpu.sync_copy(denloc.at[pl.ds(k * 128, 128)],
                                den_s.at[iot.at[k]], add=True)

        plsc.subcore_barrier()

        nout = _RPT // 128
        pltpu.async_copy(acc_s.at[pl.ds(s * _RPT, 128)], rowb.at[0], gsem[0])
        for k in range(nout):
            b = k % _NB
            b1 = (k + 1) % _NB
            pltpu.make_async_copy(
                acc_s.at[pl.ds(s * _RPT + k * 128, 128)], rowb.at[b],
                gsem[b]).wait()
            if k >= 2:
                b2 = (k - 2) % _NB
                pltpu.make_async_copy(
                    rowb.at[b2],
                    acc_out.at[c, r, pl.ds(s * _RPT + (k - 2) * 128, 128), :],
                    ssem[b2]).wait()
            if k < nout - 1:
                pltpu.async_copy(
                    acc_s.at[pl.ds(s * _RPT + (k + 1) * 128, 128)],
                    rowb.at[b1], gsem[b1])
            pltpu.async_copy(
                rowb.at[b],
                acc_out.at[c, r, pl.ds(s * _RPT + k * 128, 128), :], ssem[b])
        for kk in (nout - 2, nout - 1):
            b = kk % _NB
            pltpu.make_async_copy(
                rowb.at[b],
                acc_out.at[c, r, pl.ds(s * _RPT + kk * 128, 128), :],
                ssem[b]).wait()

        @pl.when(jnp.logical_and(c == 0, s == 0))
        def _():
            pltpu.sync_copy(den_s, denloc)
            pltpu.sync_copy(denloc, den_out.at[r])

        plsc.subcore_barrier()
        return 0

    lax.fori_loop(0, 9, _rel, 0)


def _stage2(src_p, dst_p, asrc_p, adst_p, h_p):
    mesh = plsc.VectorSubcoreMesh(core_axis_name="c", subcore_axis_name="s",
                                  num_cores=_NC, num_subcores=_NS)
    return pl.kernel(
        _sc_body,
        out_type=[
            jax.ShapeDtypeStruct((_NC, 9, _NP, _DH), jnp.float32),
            jax.ShapeDtypeStruct((9, _NP // 16, 16), jnp.float32),
        ],
        mesh=mesh,
        compiler_params=pltpu.CompilerParams(needs_layout_passes=False,
                                             use_tc_tiling_on_sc=False),
        scratch_types=[
            pltpu.VMEM((_NP,), jnp.float32),
            pltpu.VMEM((_NP,), jnp.float32),
            pltpu.VMEM((_HCHUNK, _CH), jnp.int32),
            pltpu.VMEM((_HCHUNK, _CH), jnp.int32),
            pltpu.VMEM((_NB, _CH, _DH), jnp.float32),
            pltpu.VMEM((128, _DH), jnp.float32),
            pltpu.VMEM((40, 16), jnp.float32),
            pltpu.VMEM((_NP // 16, 16), jnp.float32),
            pltpu.VMEM((5, 128), jnp.int32),
            pltpu.VMEM((16,), jnp.float32),
            pltpu.VMEM_SHARED((_NP, _DH), jnp.float32),
            pltpu.VMEM_SHARED((_NP // 16, 16), jnp.float32),
            pltpu.SemaphoreType.DMA,
            pltpu.SemaphoreType.DMA,
            pltpu.SemaphoreType.DMA,
            pltpu.SemaphoreType.DMA,
            pltpu.SemaphoreType.DMA,
            pltpu.SemaphoreType.DMA,
            pltpu.SemaphoreType.DMA,
            pltpu.SemaphoreType.DMA,
            pltpu.SemaphoreType.DMA,
        ],
    )(src_p, dst_p, asrc_p, adst_p, h_p)


def _comb_body(acc_ref, den_ref, bias_ref, op_ref, ol_ref, og_ref):
    acc = acc_ref[...]
    dn = den_ref[...]
    bias = bias_ref[...]

    def _mix(rels):
        out = 0.0
        for r in rels:
            ar = jnp.concatenate([acc[0, r], acc[1, r]], axis=-1)
            out = out + ar / (dn[:, r][:, None] + 1e-16) + bias[r][None, :]
        return jnp.maximum(out * (1.0 / 3.0), 0.0)

    op_ref[...] = _mix([0, 4, 7])
    ol_ref[...] = _mix([1, 3, 8])
    og_ref[...] = _mix([2, 5, 6])


def _stage3(acc, den, bias):
    return pl.pallas_call(
        _comb_body,
        grid=(_N // _ROWB,),
        in_specs=[
            pl.BlockSpec((_NC, 9, _ROWB, _DH), lambda i: (0, 0, i, 0)),
            pl.BlockSpec((_ROWB, 9), lambda i: (i, 0)),
            pl.BlockSpec((9, _DO), lambda i: (0, 0)),
        ],
        out_specs=[
            pl.BlockSpec((_ROWB, _DO), lambda i: (i, 0)),
            pl.BlockSpec((_ROWB, _DO), lambda i: (i, 0)),
            pl.BlockSpec((_ROWB, _DO), lambda i: (i, 0)),
        ],
        out_shape=[jax.ShapeDtypeStruct((_N, _DO), jnp.float32)] * 3,
    )(acc, den, bias)


def kernel(x_point, x_line, x_polygon, params,
           ei_point_point, ei_point_line, ei_point_polygon,
           ei_line_line, ei_line_point, ei_line_polygon,
           ei_polygon_polygon, ei_polygon_point, ei_polygon_line):
    eis = {("point", "point"): ei_point_point,
           ("point", "line"): ei_point_line,
           ("point", "polygon"): ei_point_polygon,
           ("line", "line"): ei_line_line,
           ("line", "point"): ei_line_point,
           ("line", "polygon"): ei_line_polygon,
           ("polygon", "polygon"): ei_polygon_polygon,
           ("polygon", "point"): ei_polygon_point,
           ("polygon", "line"): ei_polygon_line}
    xd = {"point": x_point, "line": x_line, "polygon": x_polygon}

    src_rels = {t: [i for i, (sr, _) in enumerate(_RELS) if sr == t] for t in _TYPES}
    dst_rels = {t: [i for i, (_, dr) in enumerate(_RELS) if dr == t] for t in _TYPES}

    wfs, uss = [], []
    for t in _TYPES:
        cols = [params["%s__%s" % _RELS[r]]["W"] for r in src_rels[t]]
        wcat = jnp.concatenate(cols, axis=1)
        wfs.append(jnp.transpose(wcat.reshape(_D, 6, _DH), (1, 0, 2)))
        ucols = [(params["%s__%s" % _RELS[r]]["W"]
                  @ params["%s__%s" % _RELS[r]]["att_src"])[:, None]
                 for r in src_rels[t]]
        ucols += [(params["%s__%s" % _RELS[r]]["W"]
                   @ params["%s__%s" % _RELS[r]]["att_dst"])[:, None]
                  for r in dst_rels[t]]
        u = jnp.concatenate(ucols, axis=1)
        uss.append(jnp.pad(u, ((0, 0), (0, _DO - u.shape[1]))))
    xs = jnp.stack([xd[t] for t in _TYPES])

    h_p = _stage1_h(xs, jnp.stack(wfs))
    a_all = _stage1_a(xs, jnp.stack(uss))

    ti = {t: i for i, t in enumerate(_TYPES)}
    asrc_list, adst_list = [None] * 9, [None] * 9
    for t in _TYPES:
        for j, r in enumerate(src_rels[t]):
            asrc_list[r] = a_all[ti[t], :, j]
        for j, r in enumerate(dst_rels[t]):
            adst_list[r] = a_all[ti[t], :, 3 + j]

    pad_n = _NP - _N
    asrc_p = jnp.stack([jnp.pad(a, (0, pad_n), constant_values=-1e30)
                        for a in asrc_list])
    adst_p = jnp.stack([jnp.pad(a, (0, pad_n)) for a in adst_list])
    src_p = jnp.stack([jnp.pad(eis[rel][0], (0, _EP - _E), constant_values=_N)
                       for rel in _RELS]).reshape(9, _NS, _NCHUNK, _CH)
    dst_p = jnp.stack([jnp.pad(eis[rel][1], (0, _EP - _E), constant_values=_N)
                       for rel in _RELS]).reshape(9, _NS, _NCHUNK, _CH)

    acc, den = _stage2(src_p, dst_p, asrc_p.reshape(-1), adst_p.reshape(-1),
                       h_p)

    bias = jnp.stack([params["%s__%s" % rel]["bias"] for rel in _RELS])
    return _stage3(acc, den.reshape(9, _NP).T, bias)


# speedup vs baseline: 1.1369x; 1.0146x over previous
import jax
import jax.numpy as jnp
from jax import lax
from jax.experimental import pallas as pl
from jax.experimental.pallas import tpu as pltpu
from jax.experimental.pallas import tpu_sc as plsc

_RELS = [("point", "point"), ("point", "line"), ("point", "polygon"),
         ("line", "line"), ("line", "point"), ("line", "polygon"),
         ("polygon", "polygon"), ("polygon", "point"), ("polygon", "line")]
_TYPES = ["point", "line", "polygon"]
_N = 10000
_NP = 10240
_E = 160000
_EP = 163840
_D = 512
_DO = 128
_DH = 64
_NC = 2
_NS = 16
_EPT = _EP // _NS
_CH = 128
_NCHUNK = _EPT // _CH
_HCHUNK = _NCHUNK // 2
_NB = 4
_RPT = _NP // _NS
_ROWB = 2000


def _mm_h_body(x_ref, w_ref, h_ref):
    h_ref[0, 0] = jnp.dot(x_ref[0], w_ref[0, 0],
                          preferred_element_type=jnp.float32)


def _stage1_h(xs, wf):
    return pl.pallas_call(
        _mm_h_body,
        grid=(3, _N // _ROWB, 3, _NC),
        in_specs=[
            pl.BlockSpec((1, _ROWB, _D), lambda t, i, j, c: (t, i, 0)),
            pl.BlockSpec((1, 1, _D, _DH),
                         lambda t, i, j, c: (t, 2 * j + c, 0, 0)),
        ],
        out_specs=pl.BlockSpec((1, 1, _ROWB, _DH),
                               lambda t, i, j, c: (c, 3 * t + j, i, 0)),
        out_shape=jax.ShapeDtypeStruct((_NC, 9, _NP, _DH), jnp.float32),
    )(xs, wf)


def _mm_a_body(x_ref, u_ref, a_ref):
    a_ref[0] = jnp.dot(x_ref[0], u_ref[0], preferred_element_type=jnp.float32)


def _stage1_a(xs, us):
    return pl.pallas_call(
        _mm_a_body,
        grid=(3, _N // _ROWB),
        in_specs=[
            pl.BlockSpec((1, _ROWB, _D), lambda t, i: (t, i, 0)),
            pl.BlockSpec((1, _D, _DO), lambda t, i: (t, 0, 0)),
        ],
        out_specs=pl.BlockSpec((1, _ROWB, _DO), lambda t, i: (t, i, 0)),
        out_shape=jax.ShapeDtypeStruct((3, _N, _DO), jnp.float32),
    )(xs, us)


def _sc_body(src_h, dst_h, asrc_h, adst_h, h_h, acc_out, den_out,
             asrc_v, adst_v, srcb, dstb, rowb, exb, zbuf, zden, denloc, iot,
             mbuf, acc_s, den_s, zsem, gs0, gs1, gs2, gs3,
             ss0, ss1, ss2, ss3):
    c = lax.axis_index("c")
    s = lax.axis_index("s")
    gsem = [gs0, gs1, gs2, gs3]
    ssem = [ss0, ss1, ss2, ss3]
    zvec = jnp.zeros((16,), jnp.float32)
    lane = lax.iota(jnp.int32, 16)

    def _zb(i, _):
        for q in range(4):
            zbuf[i, pl.ds(q * 16, 16)] = zvec
        return 0
    lax.fori_loop(0, 128, _zb, 0)

    def _zd(i, _):
        zden[i, :] = zvec
        return 0
    lax.fori_loop(0, 40, _zd, 0)

    def _io(i, _):
        k = i // 8
        j = i % 8
        iot[k, pl.ds(j * 16, 16)] = lane + i * 16
        return 0
    lax.fori_loop(0, 40, _io, 0)

    def _rel(r, _carry):
        pltpu.sync_copy(asrc_h.at[pl.ds(r * _NP, _NP)], asrc_v)
        pltpu.sync_copy(adst_h.at[pl.ds(r * _NP, _NP)], adst_v)

        zcps = [pltpu.async_copy(
            zbuf, acc_s.at[pl.ds(s * _RPT + k * 128, 128)], zsem)
            for k in range(_RPT // 128)]
        for cp in zcps:
            cp.wait()
        pltpu.sync_copy(zden, den_s.at[pl.ds(s * 40, 40)])

        def _zl(i, _):
            denloc[i, :] = zvec
            return 0
        lax.fori_loop(0, _NP // 16, _zl, 0)

        def _mxs(i, m):
            return jnp.maximum(m, asrc_v[pl.ds(i * 16, 16)])
        def _mxd(i, m):
            return jnp.maximum(m, adst_v[pl.ds(i * 16, 16)])
        neg = jnp.full((16,), -1e30, jnp.float32)

        def _lanemax(m):
            for sh in (1, 2, 4, 8):
                mbuf[...] = m
                m = jnp.maximum(m, plsc.load_gather(
                    mbuf, [jnp.bitwise_xor(lane, sh)]))
            return m
        mtot = _lanemax(lax.fori_loop(0, _NP // 16, _mxs, neg)) + \
               _lanemax(lax.fori_loop(0, _NP // 16, _mxd, neg))
        m_sh = jnp.where(mtot >= 0, mtot, 0.2 * mtot)

        plsc.subcore_barrier()

        def _ex(k, eb):
            for i in range(8):
                sv = srcb[k, pl.ds(i * 16, 16)]
                dv = dstb[k, pl.ds(i * 16, 16)]
                a = plsc.load_gather(asrc_v, [sv]) + \
                    plsc.load_gather(adst_v, [dv])
                a = jnp.where(a >= 0, a, 0.2 * a)
                ev = jnp.exp(a - m_sh)
                plsc.addupdate_scatter(
                    denloc,
                    [jnp.right_shift(dv, 4), jnp.bitwise_and(dv, 15)], ev)
                exb[eb, pl.ds(i * 16, 16)] = ev

        def _scale(b, eb):
            for i in range(8):
                ev = exb[eb, pl.ds(i * 16, 16)]
                for j2 in range(16):
                    e = ev[j2]
                    ro = i * 16 + j2
                    for q in range(4):
                        sl = pl.ds(q * 16, 16)
                        rowb[b, ro, sl] = rowb[b, ro, sl] * e

        def _half(h, _):
            pltpu.sync_copy(src_h.at[r, s, pl.ds(h * _HCHUNK, _HCHUNK)], srcb)
            pltpu.sync_copy(dst_h.at[r, s, pl.ds(h * _HCHUNK, _HCHUNK)], dstb)

            pltpu.async_copy(h_h.at[c, r].at[srcb.at[0]], rowb.at[0], gsem[0])
            _ex(0, 0)

            def _chunk(k, _k):
                bk = lax.rem(k, _NB)
                bk1 = lax.rem(k + 1, _NB)
                ek = lax.rem(k, 2)
                ek1 = lax.rem(k + 1, 2)

                @pl.when(k >= _NB - 1)
                def _():
                    for b in range(_NB):
                        @pl.when(bk1 == b)
                        def _():
                            pltpu.make_async_copy(
                                rowb.at[b], acc_s.at[dstb.at[k]],
                                ssem[b]).wait()

                @pl.when(k < _HCHUNK - 1)
                def _():
                    for b in range(_NB):
                        @pl.when(bk1 == b)
                        def _():
                            pltpu.async_copy(
                                h_h.at[c, r].at[srcb.at[k + 1]], rowb.at[b],
                                gsem[b])
                    _ex(k + 1, ek1)
                for b in range(_NB):
                    @pl.when(bk == b)
                    def _():
                        pltpu.make_async_copy(
                            h_h.at[c, r].at[srcb.at[k]], rowb.at[b],
                            gsem[b]).wait()
                _scale(bk, ek)
                for b in range(_NB):
                    @pl.when(bk == b)
                    def _():
                        pltpu.async_copy(
                            rowb.at[b], acc_s.at[dstb.at[k]], ssem[b],
                            add=True)
                return 0
            lax.fori_loop(0, _HCHUNK, _chunk, 0)
            for kk in range(_HCHUNK - _NB + 1, _HCHUNK):
                b = kk % _NB
                pltpu.make_async_copy(
                    rowb.at[b], acc_s.at[dstb.at[kk]], ssem[b]).wait()
            return 0
        lax.fori_loop(0, 2, _half, 0)

        @pl.when(c == 0)
        def _():
            for k in range(5):
                pltpu.sync_copy(denloc.at[pl.ds(k * 128, 128)],
                                den_s.at[iot.at[k]], add=True)

        plsc.subcore_barrier()

        nout = _RPT // 128
        pltpu.async_copy(acc_s.at[pl.ds(s * _RPT, 128)], rowb.at[0], gsem[0])
        for k in range(nout):
            b = k % _NB
            b1 = (k + 1) % _NB
            pltpu.make_async_copy(
                acc_s.at[pl.ds(s * _RPT + k * 128, 128)], rowb.at[b],
                gsem[b]).wait()
            if k >= 2:
                b2 = (k - 2) % _NB
                pltpu.make_async_copy(
                    rowb.at[b2],
                    acc_out.at[c, r, pl.ds(s * _RPT + (k - 2) * 128, 128), :],
                    ssem[b2]).wait()
            if k < nout - 1:
                pltpu.async_copy(
                    acc_s.at[pl.ds(s * _RPT + (k + 1) * 128, 128)],
                    rowb.at[b1], gsem[b1])
            pltpu.async_copy(
                rowb.at[b],
                acc_out.at[c, r, pl.ds(s * _RPT + k * 128, 128), :], ssem[b])
        for kk in (nout - 2, nout - 1):
            b = kk % _NB
            pltpu.make_async_copy(
                rowb.at[b],
                acc_out.at[c, r, pl.ds(s * _RPT + kk * 128, 128), :],
                ssem[b]).wait()

        @pl.when(jnp.logical_and(c == 0, s == 0))
        def _():
            pltpu.sync_copy(den_s, denloc)
            pltpu.sync_copy(denloc, den_out.at[r])

        plsc.subcore_barrier()
        return 0

    lax.fori_loop(0, 9, _rel, 0)


def _stage2(src_p, dst_p, asrc_p, adst_p, h_p):
    mesh = plsc.VectorSubcoreMesh(core_axis_name="c", subcore_axis_name="s",
                                  num_cores=_NC, num_subcores=_NS)
    return pl.kernel(
        _sc_body,
        out_type=[
            jax.ShapeDtypeStruct((_NC, 9, _NP, _DH), jnp.float32),
            jax.ShapeDtypeStruct((9, _NP // 16, 16), jnp.float32),
        ],
        mesh=mesh,
        compiler_params=pltpu.CompilerParams(needs_layout_passes=False,
                                             use_tc_tiling_on_sc=False),
        scratch_types=[
            pltpu.VMEM((_NP,), jnp.float32),
            pltpu.VMEM((_NP,), jnp.float32),
            pltpu.VMEM((_HCHUNK, _CH), jnp.int32),
            pltpu.VMEM((_HCHUNK, _CH), jnp.int32),
            pltpu.VMEM((_NB, _CH, _DH), jnp.float32),
            pltpu.VMEM((2, _CH), jnp.float32),
            pltpu.VMEM((128, _DH), jnp.float32),
            pltpu.VMEM((40, 16), jnp.float32),
            pltpu.VMEM((_NP // 16, 16), jnp.float32),
            pltpu.VMEM((5, 128), jnp.int32),
            pltpu.VMEM((16,), jnp.float32),
            pltpu.VMEM_SHARED((_NP, _DH), jnp.float32),
            pltpu.VMEM_SHARED((_NP // 16, 16), jnp.float32),
            pltpu.SemaphoreType.DMA,
            pltpu.SemaphoreType.DMA,
            pltpu.SemaphoreType.DMA,
            pltpu.SemaphoreType.DMA,
            pltpu.SemaphoreType.DMA,
            pltpu.SemaphoreType.DMA,
            pltpu.SemaphoreType.DMA,
            pltpu.SemaphoreType.DMA,
            pltpu.SemaphoreType.DMA,
        ],
    )(src_p, dst_p, asrc_p, adst_p, h_p)


def _comb_body(acc_ref, den_ref, bias_ref, op_ref, ol_ref, og_ref):
    acc = acc_ref[...]
    dn = den_ref[...]
    bias = bias_ref[...]

    def _mix(rels):
        out = 0.0
        for r in rels:
            ar = jnp.concatenate([acc[0, r], acc[1, r]], axis=-1)
            out = out + ar / (dn[:, r][:, None] + 1e-16) + bias[r][None, :]
        return jnp.maximum(out * (1.0 / 3.0), 0.0)

    op_ref[...] = _mix([0, 4, 7])
    ol_ref[...] = _mix([1, 3, 8])
    og_ref[...] = _mix([2, 5, 6])


def _stage3(acc, den, bias):
    return pl.pallas_call(
        _comb_body,
        grid=(_N // _ROWB,),
        in_specs=[
            pl.BlockSpec((_NC, 9, _ROWB, _DH), lambda i: (0, 0, i, 0)),
            pl.BlockSpec((_ROWB, 9), lambda i: (i, 0)),
            pl.BlockSpec((9, _DO), lambda i: (0, 0)),
        ],
        out_specs=[
            pl.BlockSpec((_ROWB, _DO), lambda i: (i, 0)),
            pl.BlockSpec((_ROWB, _DO), lambda i: (i, 0)),
            pl.BlockSpec((_ROWB, _DO), lambda i: (i, 0)),
        ],
        out_shape=[jax.ShapeDtypeStruct((_N, _DO), jnp.float32)] * 3,
    )(acc, den, bias)


def kernel(x_point, x_line, x_polygon, params,
           ei_point_point, ei_point_line, ei_point_polygon,
           ei_line_line, ei_line_point, ei_line_polygon,
           ei_polygon_polygon, ei_polygon_point, ei_polygon_line):
    eis = {("point", "point"): ei_point_point,
           ("point", "line"): ei_point_line,
           ("point", "polygon"): ei_point_polygon,
           ("line", "line"): ei_line_line,
           ("line", "point"): ei_line_point,
           ("line", "polygon"): ei_line_polygon,
           ("polygon", "polygon"): ei_polygon_polygon,
           ("polygon", "point"): ei_polygon_point,
           ("polygon", "line"): ei_polygon_line}
    xd = {"point": x_point, "line": x_line, "polygon": x_polygon}

    src_rels = {t: [i for i, (sr, _) in enumerate(_RELS) if sr == t] for t in _TYPES}
    dst_rels = {t: [i for i, (_, dr) in enumerate(_RELS) if dr == t] for t in _TYPES}

    wfs, uss = [], []
    for t in _TYPES:
        cols = [params["%s__%s" % _RELS[r]]["W"] for r in src_rels[t]]
        wcat = jnp.concatenate(cols, axis=1)
        wfs.append(jnp.transpose(wcat.reshape(_D, 6, _DH), (1, 0, 2)))
        ucols = [(params["%s__%s" % _RELS[r]]["W"]
                  @ params["%s__%s" % _RELS[r]]["att_src"])[:, None]
                 for r in src_rels[t]]
        ucols += [(params["%s__%s" % _RELS[r]]["W"]
                   @ params["%s__%s" % _RELS[r]]["att_dst"])[:, None]
                  for r in dst_rels[t]]
        u = jnp.concatenate(ucols, axis=1)
        uss.append(jnp.pad(u, ((0, 0), (0, _DO - u.shape[1]))))
    xs = jnp.stack([xd[t] for t in _TYPES])

    h_p = _stage1_h(xs, jnp.stack(wfs))
    a_all = _stage1_a(xs, jnp.stack(uss))

    ti = {t: i for i, t in enumerate(_TYPES)}
    asrc_list, adst_list = [None] * 9, [None] * 9
    for t in _TYPES:
        for j, r in enumerate(src_rels[t]):
            asrc_list[r] = a_all[ti[t], :, j]
        for j, r in enumerate(dst_rels[t]):
            adst_list[r] = a_all[ti[t], :, 3 + j]

    pad_n = _NP - _N
    asrc_p = jnp.stack([jnp.pad(a, (0, pad_n), constant_values=-1e30)
                        for a in asrc_list])
    adst_p = jnp.stack([jnp.pad(a, (0, pad_n)) for a in adst_list])
    src_p = jnp.stack([jnp.pad(eis[rel][0], (0, _EP - _E), constant_values=_N)
                       for rel in _RELS]).reshape(9, _NS, _NCHUNK, _CH)
    dst_p = jnp.stack([jnp.pad(eis[rel][1], (0, _EP - _E), constant_values=_N)
                       for rel in _RELS]).reshape(9, _NS, _NCHUNK, _CH)

    acc, den = _stage2(src_p, dst_p, asrc_p.reshape(-1), adst_p.reshape(-1),
                       h_p)

    bias = jnp.stack([params["%s__%s" % rel]["bias"] for rel in _RELS])
    return _stage3(acc, den.reshape(9, _NP).T, bias)
